# Initial kernel scaffold; baseline (speedup 1.0000x reference)
import jax
import jax.numpy as jnp
from jax.experimental import pallas as pl


def kernel(x, edge_index, edge_attr, encoded_x, batch, W1, b1, W2, b2):
    raise NotImplementedError("write your pallas kernel here")


# speedup vs baseline: 7.3920x; 7.3920x over previous
import functools

import jax
import jax.numpy as jnp
from jax import lax
from jax.experimental import pallas as pl
from jax.experimental.pallas import tpu as pltpu
from jax.experimental.pallas import tpu_sc as plsc

N_NODES = 10000
N_EDGES = 320000
D = 128

NC = 2
NS = 16
NW = NC * NS

CHUNK = 80
EDGES_PER_TILE = N_EDGES // NW
CHUNKS_PER_TILE = EDGES_PER_TILE // CHUNK
NBUF = 4
ROWS_MAIN = 624
ROWS_TAIL = N_NODES - NS * ROWS_MAIN


def _sc_segment_sum(col, edge_attr, zeros):
    mesh = plsc.VectorSubcoreMesh(core_axis_name="c", subcore_axis_name="s")

    @functools.partial(
        pl.kernel,
        mesh=mesh,
        out_type=jax.ShapeDtypeStruct((NC, N_NODES, D), jnp.float32),
        scratch_types=[
            pltpu.VMEM_SHARED((N_NODES, D), jnp.float32),
            pltpu.VMEM((NBUF, CHUNK, D), jnp.float32),
            pltpu.VMEM((NBUF, CHUNK), jnp.int32),
            *([pltpu.SemaphoreType.DMA] * NBUF),
        ],
    )
    def seg_sum(col_hbm, ea_hbm, z_hbm, out_hbm, acc, rows, idx, *sems):
        c = lax.axis_index("c")
        s = lax.axis_index("s")
        w = c * NS + s
        tile_base = w * EDGES_PER_TILE

        r0 = s * ROWS_MAIN
        pltpu.sync_copy(z_hbm.at[pl.ds(r0, ROWS_MAIN)],
                        acc.at[pl.ds(r0, ROWS_MAIN)])

        @pl.when(s == NS - 1)
        def _zero_tail():
            pltpu.sync_copy(z_hbm.at[pl.ds(NS * ROWS_MAIN, ROWS_TAIL)],
                            acc.at[pl.ds(NS * ROWS_MAIN, ROWS_TAIL)])

        plsc.subcore_barrier()

        def rows_copy(t, b):
            off = tile_base + t * CHUNK
            return pltpu.make_async_copy(
                ea_hbm.at[pl.ds(off, CHUNK)], rows.at[b], sems[b])

        def idx_copy(t, b):
            off = tile_base + t * CHUNK
            return pltpu.make_async_copy(
                col_hbm.at[pl.ds(off, CHUNK)], idx.at[b], sems[b])

        def drain_and_scatter(t, b):
            rows_copy(t, b).wait()
            idx_copy(t, b).wait()
            pltpu.sync_copy(rows.at[b], acc.at[idx.at[b]], add=True)

        full_groups = CHUNKS_PER_TILE // NBUF
        rem = CHUNKS_PER_TILE - full_groups * NBUF

        for b in range(NBUF):
            rows_copy(b, b).start()
            idx_copy(b, b).start()

        def body(g, carry):
            for b in range(NBUF):
                t = g * NBUF + b
                drain_and_scatter(t, b)
                rows_copy(t + NBUF, b).start()
                idx_copy(t + NBUF, b).start()
            return carry

        lax.fori_loop(0, full_groups - 1, body, 0)
        for b in range(NBUF):
            drain_and_scatter((full_groups - 1) * NBUF + b, b)
        for r in range(rem):
            t = full_groups * NBUF + r
            rows_copy(t, 0).start()
            idx_copy(t, 0).start()
            drain_and_scatter(t, 0)

        plsc.subcore_barrier()
        pltpu.sync_copy(acc.at[pl.ds(r0, ROWS_MAIN)],
                        out_hbm.at[c, pl.ds(r0, ROWS_MAIN)])

        @pl.when(s == NS - 1)
        def _write_tail():
            pltpu.sync_copy(acc.at[pl.ds(NS * ROWS_MAIN, ROWS_TAIL)],
                            out_hbm.at[c, pl.ds(NS * ROWS_MAIN, ROWS_TAIL)])

    return seg_sum(col, edge_attr, zeros)


ROW_BLK = 1000


def _tc_mlp_body(x_ref, e_ref, p0_ref, p1_ref, w1_ref, b1_ref, w2_ref, b2_ref,
                 o_ref):
    recv = p0_ref[...] + p1_ref[...]
    h = (x_ref[...] @ w1_ref[0:D, :]
         + e_ref[...] @ w1_ref[D:2 * D, :]
         + recv @ w1_ref[2 * D:3 * D, :]
         + b1_ref[...])
    h = jnp.maximum(h, 0.0)
    o = h @ w2_ref[...] + b2_ref[...]
    o_ref[...] = jnp.maximum(o, 0.0)


def _tc_mlp(x, encoded_x, p0, p1, W1, b1, W2, b2):
    n = x.shape[0]
    grid = (n // ROW_BLK,)
    row_spec = pl.BlockSpec((ROW_BLK, D), lambda i: (i, 0))
    full = lambda shape: pl.BlockSpec(shape, lambda i: (0,) * len(shape))
    return pl.pallas_call(
        _tc_mlp_body,
        grid=grid,
        in_specs=[row_spec, row_spec, row_spec, row_spec,
                  full((3 * D, D)), full((1, D)), full((D, D)), full((1, D))],
        out_specs=row_spec,
        out_shape=jax.ShapeDtypeStruct((n, D), jnp.float32),
    )(x, encoded_x, p0, p1, W1, b1.reshape(1, D), W2, b2.reshape(1, D))


def kernel(x, edge_index, edge_attr, encoded_x, batch, W1, b1, W2, b2):
    col = edge_index[1].astype(jnp.int32)
    zeros = jnp.zeros((N_NODES, D), jnp.float32)
    partials = _sc_segment_sum(col, edge_attr, zeros)
    return _tc_mlp(x, encoded_x, partials[0], partials[1], W1, b1, W2, b2)


# speedup vs baseline: 8.6230x; 1.1665x over previous
import functools

import jax
import jax.numpy as jnp
from jax import lax
from jax.experimental import pallas as pl
from jax.experimental.pallas import tpu as pltpu
from jax.experimental.pallas import tpu_sc as plsc

N_NODES = 10000
N_EDGES = 320000
D = 128

NC = 2
NS = 16
NW = NC * NS

CHUNK = 128
EDGES_MAIN = 9984
FULL_CHUNKS = EDGES_MAIN // CHUNK
TAIL_CHUNKS = (N_EDGES - NW * EDGES_MAIN) // CHUNK
NBUF = 3
ROWS_MAIN = 624
ROWS_TAIL = N_NODES - NS * ROWS_MAIN


def _sc_segment_sum(edge_index, edge_attr, zeros):
    mesh = plsc.VectorSubcoreMesh(core_axis_name="c", subcore_axis_name="s")

    @functools.partial(
        pl.kernel,
        mesh=mesh,
        out_type=jax.ShapeDtypeStruct((NC, N_NODES, D), jnp.float32),
        scratch_types=[
            pltpu.VMEM_SHARED((N_NODES, D), jnp.float32),
            pltpu.VMEM((NBUF, CHUNK, D), jnp.float32),
            pltpu.VMEM((NBUF, 2, CHUNK), jnp.int32),
            *([pltpu.SemaphoreType.DMA] * NBUF),
        ],
    )
    def seg_sum(ei_hbm, ea_hbm, z_hbm, out_hbm, acc, rows, idx, *sems):
        c = lax.axis_index("c")
        s = lax.axis_index("s")
        w = c * NS + s
        tile_base = w * EDGES_MAIN

        r0 = s * ROWS_MAIN
        pltpu.sync_copy(z_hbm.at[pl.ds(r0, ROWS_MAIN)],
                        acc.at[pl.ds(r0, ROWS_MAIN)])

        @pl.when(s == NS - 1)
        def _zero_tail():
            pltpu.sync_copy(z_hbm.at[pl.ds(NS * ROWS_MAIN, ROWS_TAIL)],
                            acc.at[pl.ds(NS * ROWS_MAIN, ROWS_TAIL)])

        plsc.subcore_barrier()

        def rows_copy(off, b):
            return pltpu.make_async_copy(
                ea_hbm.at[pl.ds(off, CHUNK)], rows.at[b], sems[b])

        def idx_copy(off, b):
            return pltpu.make_async_copy(
                ei_hbm.at[pl.ds(0, 2), pl.ds(off, CHUNK)], idx.at[b], sems[b])

        def start(off, b):
            rows_copy(off, b).start()
            idx_copy(off, b).start()

        def drain_and_scatter(off, b):
            rows_copy(off, b).wait()
            idx_copy(off, b).wait()
            pltpu.sync_copy(rows.at[b], acc.at[idx.at[b, 1]], add=True)

        for b in range(NBUF):
            start(tile_base + b * CHUNK, b)

        def body(g, carry):
            for b in range(NBUF):
                t = g * NBUF + b
                off = tile_base + t * CHUNK
                drain_and_scatter(off, b)
                start(off + NBUF * CHUNK, b)
            return carry

        groups = FULL_CHUNKS // NBUF
        lax.fori_loop(0, groups - 1, body, 0)
        for b in range(NBUF):
            drain_and_scatter(tile_base + ((groups - 1) * NBUF + b) * CHUNK, b)

        @pl.when(w < TAIL_CHUNKS)
        def _tail_chunk():
            off = NW * EDGES_MAIN + w * CHUNK
            start(off, 0)
            drain_and_scatter(off, 0)

        plsc.subcore_barrier()
        pltpu.sync_copy(acc.at[pl.ds(r0, ROWS_MAIN)],
                        out_hbm.at[c, pl.ds(r0, ROWS_MAIN)])

        @pl.when(s == NS - 1)
        def _write_tail():
            pltpu.sync_copy(acc.at[pl.ds(NS * ROWS_MAIN, ROWS_TAIL)],
                            out_hbm.at[c, pl.ds(NS * ROWS_MAIN, ROWS_TAIL)])

    return seg_sum(edge_index, edge_attr, zeros)


ROW_BLK = 1000


def _tc_pre_body(x_ref, e_ref, w1_ref, b1_ref, o_ref):
    o_ref[...] = (x_ref[...] @ w1_ref[0:D, :]
                  + e_ref[...] @ w1_ref[D:2 * D, :]
                  + b1_ref[...])


def _tc_pre(x, encoded_x, W1, b1):
    n = x.shape[0]
    row_spec = pl.BlockSpec((ROW_BLK, D), lambda i: (i, 0))
    full = lambda shape: pl.BlockSpec(shape, lambda i: (0,) * len(shape))
    return pl.pallas_call(
        _tc_pre_body,
        grid=(n // ROW_BLK,),
        in_specs=[row_spec, row_spec, full((3 * D, D)), full((1, D))],
        out_specs=row_spec,
        out_shape=jax.ShapeDtypeStruct((n, D), jnp.float32),
    )(x, encoded_x, W1, b1.reshape(1, D))


def _tc_post_body(pre_ref, p_ref, w1_ref, w2_ref, b2_ref, o_ref):
    recv = p_ref[0] + p_ref[1]
    h = jnp.maximum(pre_ref[...] + recv @ w1_ref[2 * D:3 * D, :], 0.0)
    o = h @ w2_ref[...] + b2_ref[...]
    o_ref[...] = jnp.maximum(o, 0.0)


def _tc_post(pre, partials, W1, W2, b2):
    n = pre.shape[0]
    row_spec = pl.BlockSpec((ROW_BLK, D), lambda i: (i, 0))
    full = lambda shape: pl.BlockSpec(shape, lambda i: (0,) * len(shape))
    p_spec = pl.BlockSpec((NC, ROW_BLK, D), lambda i: (0, i, 0))
    return pl.pallas_call(
        _tc_post_body,
        grid=(n // ROW_BLK,),
        in_specs=[row_spec, p_spec, full((3 * D, D)), full((D, D)), full((1, D))],
        out_specs=row_spec,
        out_shape=jax.ShapeDtypeStruct((n, D), jnp.float32),
    )(pre, partials, W1, W2, b2.reshape(1, D))


def kernel(x, edge_index, edge_attr, encoded_x, batch, W1, b1, W2, b2):
    edge_index = edge_index.astype(jnp.int32)
    zeros = jnp.zeros((N_NODES, D), jnp.float32)
    partials = _sc_segment_sum(edge_index, edge_attr, zeros)
    pre = _tc_pre(x, encoded_x, W1, b1)
    return _tc_post(pre, partials, W1, W2, b2)
